# baseline (speedup 1.0000x reference)
import functools

import jax
import jax.numpy as jnp
from jax import lax
from jax.experimental import pallas as pl
from jax.experimental.pallas import tpu as pltpu
from jax.experimental.pallas import tpu_sc as plsc

NC = 2
NS = 16
NW = NC * NS
LANES = 16
CHUNK = 96
ZROWS = 64


def _pad_rows(n_nodes: int, d: int) -> int:
    unit = max(NS * 8, d)
    per = -(-n_nodes // unit) * unit
    if per == n_nodes:
        per += unit
    assert per % d == 0 and (per // NS) % 8 == 0
    return per


@functools.cache
def _segsum_fn(n_nodes: int, d: int, nchunk: int):
    assert nchunk % 2 == 1 and nchunk >= 3
    n_pad = _pad_rows(n_nodes, d)
    rpt = n_pad // NS
    crows = n_pad // d
    assert d & (d - 1) == 0
    dshift = d.bit_length() - 1

    mesh = plsc.VectorSubcoreMesh(
        core_axis_name="c", subcore_axis_name="s",
        num_cores=NC, num_subcores=NS)

    @functools.partial(
        pl.kernel,
        compiler_params=pltpu.CompilerParams(needs_layout_passes=False),
        out_type=(
            jax.ShapeDtypeStruct((NC, n_pad, d), jnp.float32),
            jax.ShapeDtypeStruct((NC, crows, d), jnp.int32),
        ),
        mesh=mesh,
        scratch_types=[
            pltpu.VMEM_SHARED((n_pad, d), jnp.float32),
            pltpu.VMEM_SHARED((crows, d), jnp.int32),
            pltpu.VMEM((CHUNK,), jnp.int32),
            pltpu.VMEM((CHUNK,), jnp.int32),
            pltpu.VMEM((nchunk, CHUNK), jnp.int32),
            pltpu.VMEM((CHUNK, d), jnp.float32),
            pltpu.VMEM((CHUNK, d), jnp.float32),
            pltpu.VMEM((crows, d), jnp.int32),
            pltpu.VMEM((crows,), jnp.int32),
            pltpu.SemaphoreType.DMA,
            pltpu.SemaphoreType.DMA,
            pltpu.SemaphoreType.DMA,
            pltpu.SemaphoreType.DMA,
            pltpu.SemaphoreType.DMA,
            pltpu.SemaphoreType.DMA,
        ],
    )
    def segsum(x_hbm, src_hbm, dst_hbm, feat_hbm, cnt_hbm,
               acc, cacc, idx_s0, idx_s1, idx_d, rows0, rows1, cnt, idx_id,
               si0, si1, sg0, sg1, ss0, ss1):
        c = lax.axis_index("c")
        s = lax.axis_index("s")
        wid = s * NC + c
        idx_s = (idx_s0, idx_s1)
        rows = (rows0, rows1)
        si = (si0, si1)
        sg = (sg0, sg1)
        ss = (ss0, ss1)

        idx_cp_d = pltpu.async_copy(dst_hbm.at[wid], idx_d, ss0)

        def zrow(i, carry):
            for j in range(d // LANES):
                rows0[i, pl.ds(j * LANES, LANES)] = jnp.zeros(
                    (LANES,), jnp.float32)
            return carry
        lax.fori_loop(0, ZROWS, zrow, 0)

        def crow(i, carry):
            for j in range(d // LANES):
                cnt[i, pl.ds(j * LANES, LANES)] = jnp.zeros(
                    (LANES,), jnp.int32)
            return carry
        lax.fori_loop(0, crows, crow, 0)

        for j in range(crows // LANES):
            idx_id[pl.ds(j * LANES, LANES)] = (
                lax.iota(jnp.int32, LANES) + j * LANES)
        if crows % LANES:
            idx_id[pl.ds(crows - LANES, LANES)] = (
                lax.iota(jnp.int32, LANES) + (crows - LANES))

        for k in range(rpt // ZROWS):
            pltpu.sync_copy(rows0.at[pl.ds(0, ZROWS)],
                            acc.at[pl.ds(s * rpt + k * ZROWS, ZROWS)])
        ztail = rpt % ZROWS
        if ztail:
            pltpu.sync_copy(
                rows0.at[pl.ds(0, ztail)],
                acc.at[pl.ds(s * rpt + rpt - ztail, ztail)])
        @pl.when(s == 0)
        def _clear_cacc():
            pltpu.sync_copy(cnt, cacc)
        idx_cp_d.wait()
        plsc.subcore_barrier()

        ones16 = jnp.ones((LANES,), jnp.int32)
        last = nchunk - 1

        def start_idx(j, p):
            jc = jnp.minimum(j, last)
            pltpu.async_copy(src_hbm.at[wid, jc], idx_s[p], si[p])

        def wait_idx(p):
            pltpu.make_async_copy(src_hbm.at[0, 0], idx_s[p], si[p]).wait()

        def start_gather(j, p):
            del j
            pltpu.async_copy(x_hbm.at[idx_s[p]], rows[p], sg[p])

        def wait_gather(p):
            pltpu.make_async_copy(x_hbm.at[idx_s[p]], rows[p],
                                  sg[p]).wait()

        def start_scatter(j, p):
            pltpu.async_copy(rows[p], acc.at[idx_d.at[j]], ss[p], add=True)

        def wait_scatter(p):
            pltpu.make_async_copy(rows[p], acc.at[idx_d.at[0]],
                                  ss[p]).wait()

        def hist(j):
            for v in range(CHUNK // LANES):
                dv = idx_d[j, pl.ds(v * LANES, LANES)]
                plsc.addupdate_scatter(
                    cnt,
                    [lax.shift_right_logical(dv, dshift),
                     lax.bitwise_and(dv, d - 1)],
                    ones16)

        start_idx(0, 0)
        start_idx(1, 1)
        wait_idx(0)
        start_gather(0, 0)
        wait_gather(0)
        start_scatter(0, 0)
        start_idx(2, 0)
        hist(0)
        wait_idx(1)
        start_gather(1, 1)
        wait_gather(1)
        start_scatter(1, 1)
        start_idx(3, 1)
        hist(1)
        wait_scatter(0)
        wait_idx(0)
        start_gather(2, 0)

        def body(t, carry):
            j0 = 2 * t
            wait_gather(0)
            start_scatter(j0, 0)
            start_idx(j0 + 2, 0)
            hist(j0)
            wait_scatter(1)
            wait_idx(1)
            start_gather(j0 + 1, 1)
            wait_gather(1)
            start_scatter(j0 + 1, 1)
            start_idx(j0 + 3, 1)
            hist(j0 + 1)
            wait_scatter(0)
            wait_idx(0)
            start_gather(j0 + 2, 0)
            return carry
        lax.fori_loop(1, (nchunk - 1) // 2, body, 0)

        wait_gather(0)
        start_scatter(last, 0)
        hist(last)
        wait_idx(1)
        wait_scatter(1)
        wait_scatter(0)

        pltpu.sync_copy(cnt, cacc.at[idx_id], add=True)
        plsc.subcore_barrier()

        pltpu.sync_copy(acc.at[pl.ds(s * rpt, rpt)],
                        feat_hbm.at[c, pl.ds(s * rpt, rpt)])
        @pl.when(s == 0)
        def _dump_cnt():
            pltpu.sync_copy(cacc, cnt_hbm.at[c])

    return segsum


@functools.cache
def _dense_fn(n_nodes: int, d_in: int, d_out: int):
    blk = 400
    grid = n_nodes // blk

    def body(x_ref, p_ref, cnt_ref, deg_ref, wst_ref, bs_ref, wnt_ref,
             bn_ref, o_ref):
        ns = p_ref[0] + p_ref[1]
        cnt = (cnt_ref[0] + cnt_ref[1]).astype(jnp.float32)
        agg = (jnp.dot(ns, wnt_ref[...],
                       preferred_element_type=jnp.float32)
               + cnt * bn_ref[...])
        denom = jnp.maximum(deg_ref[...], 1.0)
        z = (jnp.dot(x_ref[...], wst_ref[...],
                     preferred_element_type=jnp.float32)
             + bs_ref[...] + agg / denom)
        o_ref[...] = jnp.where(z >= 0.0, z, 0.1 * z)

    return pl.pallas_call(
        body,
        grid=(grid,),
        in_specs=[
            pl.BlockSpec((blk, d_in), lambda i: (i, 0)),
            pl.BlockSpec((NC, blk, d_in), lambda i: (0, i, 0)),
            pl.BlockSpec((NC, blk, 1), lambda i: (0, i, 0)),
            pl.BlockSpec((blk, 1), lambda i: (i, 0)),
            pl.BlockSpec((d_in, d_out), lambda i: (0, 0)),
            pl.BlockSpec((1, d_out), lambda i: (0, 0)),
            pl.BlockSpec((d_in, d_out), lambda i: (0, 0)),
            pl.BlockSpec((1, d_out), lambda i: (0, 0)),
        ],
        out_specs=pl.BlockSpec((blk, d_out), lambda i: (i, 0)),
        out_shape=jax.ShapeDtypeStruct((n_nodes, d_out), jnp.float32),
    )


def kernel(x, edge_index, deg, W_self, b_self, W_neigh, b_neigh):
    b, n_nodes, d_in = x.shape
    d_out = W_neigh.shape[0]
    n_edges = edge_index.shape[1]
    epw = n_edges // NW
    assert epw * NW == n_edges
    nchunk = -(-epw // CHUNK)
    if nchunk % 2 == 0 or nchunk < 3:
        nchunk += 1 + 2 * (nchunk < 2)
    pad = nchunk * CHUNK - epw
    n_pad = _pad_rows(n_nodes, d_in)

    src = edge_index[0].astype(jnp.int32).reshape(NW, epw)
    dst = edge_index[1].astype(jnp.int32).reshape(NW, epw)
    if pad:
        src = jnp.concatenate(
            [src, jnp.zeros((NW, pad), jnp.int32)], axis=1)
        dst = jnp.concatenate(
            [dst, jnp.full((NW, pad), n_pad - 1, jnp.int32)], axis=1)
    src = src.reshape(NW, nchunk, CHUNK)
    dst = dst.reshape(NW, nchunk, CHUNK)
    deg_f = jnp.asarray(deg).astype(jnp.float32).reshape(n_nodes, 1)
    wst = W_self.astype(jnp.float32).T
    wnt = W_neigh.astype(jnp.float32).T
    bs = b_self.astype(jnp.float32).reshape(1, d_out)
    bn = b_neigh.astype(jnp.float32).reshape(1, d_out)

    segsum = _segsum_fn(n_nodes, d_in, nchunk)
    dense = _dense_fn(n_nodes, d_in, d_out)

    outs = []
    for bi in range(b):
        xb = x[bi].astype(jnp.float32)
        feat, cnt = segsum(xb, src, dst)
        cnt_n = cnt.reshape(NC, -1)[:, :n_nodes, None]
        outs.append(dense(xb, feat, cnt_n, deg_f, wst, bs, wnt, bn))
    return jnp.stack(outs, axis=0).astype(x.dtype)


# speedup vs baseline: 1.4242x; 1.4242x over previous
import functools

import jax
import jax.numpy as jnp
from jax import lax
from jax.experimental import pallas as pl
from jax.experimental.pallas import tpu as pltpu
from jax.experimental.pallas import tpu_sc as plsc

NC = 2
NS = 16
NW = NC * NS
LANES = 16
CHUNK = 96
ZROWS = 64


def _pad_rows(n_nodes: int, d: int) -> int:
    unit = max(NS * 8, d)
    per = -(-n_nodes // unit) * unit
    if per == n_nodes:
        per += unit
    assert per % d == 0 and (per // NS) % 8 == 0
    return per


@functools.cache
def _segsum_fn(n_nodes: int, d: int, nchunk: int):
    assert nchunk % 2 == 1 and nchunk >= 3
    n_pad = _pad_rows(n_nodes, d)
    rpt = n_pad // NS
    crows = n_pad // d
    assert d & (d - 1) == 0
    dshift = d.bit_length() - 1

    mesh = plsc.VectorSubcoreMesh(
        core_axis_name="c", subcore_axis_name="s",
        num_cores=NC, num_subcores=NS)

    @functools.partial(
        pl.kernel,
        compiler_params=pltpu.CompilerParams(needs_layout_passes=False),
        out_type=(
            jax.ShapeDtypeStruct((NC, n_pad, d), jnp.float32),
            jax.ShapeDtypeStruct((NC, crows, d), jnp.int32),
        ),
        mesh=mesh,
        scratch_types=[
            pltpu.VMEM_SHARED((n_pad, d), jnp.float32),
            pltpu.VMEM_SHARED((crows, d), jnp.int32),
            pltpu.VMEM((CHUNK,), jnp.int32),
            pltpu.VMEM((CHUNK,), jnp.int32),
            pltpu.VMEM((nchunk, CHUNK), jnp.int32),
            pltpu.VMEM((CHUNK, d), jnp.float32),
            pltpu.VMEM((CHUNK, d), jnp.float32),
            pltpu.VMEM((crows, d), jnp.int32),
            pltpu.VMEM((crows,), jnp.int32),
            pltpu.SemaphoreType.DMA,
            pltpu.SemaphoreType.DMA,
            pltpu.SemaphoreType.DMA,
            pltpu.SemaphoreType.DMA,
            pltpu.SemaphoreType.DMA,
            pltpu.SemaphoreType.DMA,
        ],
    )
    def segsum(x_hbm, src_hbm, dst_hbm, feat_hbm, cnt_hbm,
               acc, cacc, idx_s0, idx_s1, idx_d, rows0, rows1, cnt, idx_id,
               si0, si1, sg0, sg1, ss0, ss1):
        c = lax.axis_index("c")
        s = lax.axis_index("s")
        wid = s * NC + c
        idx_s = (idx_s0, idx_s1)
        rows = (rows0, rows1)
        si = (si0, si1)
        sg = (sg0, sg1)
        ss = (ss0, ss1)

        idx_cp_d = pltpu.async_copy(dst_hbm.at[wid], idx_d, ss0)

        def zrow(i, carry):
            for j in range(d // LANES):
                rows0[i, pl.ds(j * LANES, LANES)] = jnp.zeros(
                    (LANES,), jnp.float32)
            return carry
        lax.fori_loop(0, ZROWS, zrow, 0)

        def crow(i, carry):
            for j in range(d // LANES):
                cnt[i, pl.ds(j * LANES, LANES)] = jnp.zeros(
                    (LANES,), jnp.int32)
            return carry
        lax.fori_loop(0, crows, crow, 0)

        for j in range(crows // LANES):
            idx_id[pl.ds(j * LANES, LANES)] = (
                lax.iota(jnp.int32, LANES) + j * LANES)
        if crows % LANES:
            idx_id[pl.ds(crows - LANES, LANES)] = (
                lax.iota(jnp.int32, LANES) + (crows - LANES))

        for k in range(rpt // ZROWS):
            pltpu.sync_copy(rows0.at[pl.ds(0, ZROWS)],
                            acc.at[pl.ds(s * rpt + k * ZROWS, ZROWS)])
        ztail = rpt % ZROWS
        if ztail:
            pltpu.sync_copy(
                rows0.at[pl.ds(0, ztail)],
                acc.at[pl.ds(s * rpt + rpt - ztail, ztail)])
        @pl.when(s == 0)
        def _clear_cacc():
            pltpu.sync_copy(cnt, cacc)
        idx_cp_d.wait()
        plsc.subcore_barrier()

        ones16 = jnp.ones((LANES,), jnp.int32)
        last = nchunk - 1

        def start_idx(j, p):
            jc = jnp.minimum(j, last)
            pltpu.async_copy(src_hbm.at[wid, jc], idx_s[p], si[p])

        def wait_idx(p):
            pltpu.make_async_copy(src_hbm.at[0, 0], idx_s[p], si[p]).wait()

        def start_gather(j, p):
            del j
            pltpu.async_copy(x_hbm.at[idx_s[p]], rows[p], sg[p])

        def wait_gather(p):
            pltpu.make_async_copy(x_hbm.at[idx_s[p]], rows[p],
                                  sg[p]).wait()

        def start_scatter(j, p):
            pltpu.async_copy(rows[p], acc.at[idx_d.at[j]], ss[p], add=True)

        def wait_scatter(p):
            pltpu.make_async_copy(rows[p], acc.at[idx_d.at[0]],
                                  ss[p]).wait()

        def hist(j):
            for v in range(CHUNK // LANES):
                dv = idx_d[j, pl.ds(v * LANES, LANES)]
                plsc.addupdate_scatter(
                    cnt,
                    [lax.shift_right_logical(dv, dshift),
                     lax.bitwise_and(dv, d - 1)],
                    ones16)

        start_idx(0, 0)
        start_idx(1, 1)
        wait_idx(0)
        start_gather(0, 0)
        wait_gather(0)
        start_scatter(0, 0)
        start_idx(2, 0)
        hist(0)
        wait_idx(1)
        start_gather(1, 1)
        wait_gather(1)
        start_scatter(1, 1)
        start_idx(3, 1)
        hist(1)
        wait_scatter(0)
        wait_idx(0)
        start_gather(2, 0)

        def body(t, carry):
            j0 = 2 * t
            wait_gather(0)
            start_scatter(j0, 0)
            start_idx(j0 + 2, 0)
            hist(j0)
            wait_scatter(1)
            wait_idx(1)
            start_gather(j0 + 1, 1)
            wait_gather(1)
            start_scatter(j0 + 1, 1)
            start_idx(j0 + 3, 1)
            hist(j0 + 1)
            wait_scatter(0)
            wait_idx(0)
            start_gather(j0 + 2, 0)
            return carry
        lax.fori_loop(1, (nchunk - 1) // 2, body, 0)

        wait_gather(0)
        start_scatter(last, 0)
        hist(last)
        wait_idx(1)
        wait_scatter(1)
        wait_scatter(0)

        pltpu.sync_copy(cnt, cacc.at[idx_id], add=True)
        plsc.subcore_barrier()

        pltpu.sync_copy(acc.at[pl.ds(s * rpt, rpt)],
                        feat_hbm.at[c, pl.ds(s * rpt, rpt)])
        @pl.when(s == 0)
        def _dump_cnt():
            pltpu.sync_copy(cacc, cnt_hbm.at[c])

    return segsum


@functools.cache
def _dense_fn(n_nodes: int, d_in: int, d_out: int):
    blk = 400
    grid = n_nodes // blk

    dn = (((1,), (1,)), ((), ()))

    def body(x_ref, p_ref, cnt_ref, deg_ref, ws_ref, bs_ref, wn_ref,
             bn_ref, o_ref):
        ns = p_ref[0] + p_ref[1]
        cnt = (cnt_ref[0] + cnt_ref[1]).astype(jnp.float32)
        agg = (lax.dot_general(ns, wn_ref[...], dn,
                               preferred_element_type=jnp.float32)
               + cnt * bn_ref[...])
        denom = jnp.maximum(deg_ref[...].astype(jnp.float32), 1.0)
        z = (lax.dot_general(x_ref[...], ws_ref[...], dn,
                             preferred_element_type=jnp.float32)
             + bs_ref[...] + agg / denom)
        o_ref[...] = jnp.where(z >= 0.0, z, 0.1 * z)

    return pl.pallas_call(
        body,
        grid=(grid,),
        in_specs=[
            pl.BlockSpec((blk, d_in), lambda i: (i, 0)),
            pl.BlockSpec((NC, blk, d_in), lambda i: (0, i, 0)),
            pl.BlockSpec((NC, blk, 1), lambda i: (0, i, 0)),
            pl.BlockSpec((blk, 1), lambda i: (i, 0)),
            pl.BlockSpec((d_out, d_in), lambda i: (0, 0)),
            pl.BlockSpec((1, d_out), lambda i: (0, 0)),
            pl.BlockSpec((d_out, d_in), lambda i: (0, 0)),
            pl.BlockSpec((1, d_out), lambda i: (0, 0)),
        ],
        out_specs=pl.BlockSpec((blk, d_out), lambda i: (i, 0)),
        out_shape=jax.ShapeDtypeStruct((n_nodes, d_out), jnp.float32),
    )


def kernel(x, edge_index, deg, W_self, b_self, W_neigh, b_neigh):
    b, n_nodes, d_in = x.shape
    d_out = W_neigh.shape[0]
    n_edges = edge_index.shape[1]
    epw = n_edges // NW
    assert epw * NW == n_edges
    nchunk = -(-epw // CHUNK)
    if nchunk % 2 == 0 or nchunk < 3:
        nchunk += 1 + 2 * (nchunk < 2)
    pad = nchunk * CHUNK - epw
    n_pad = _pad_rows(n_nodes, d_in)

    src = edge_index[0].astype(jnp.int32).reshape(NW, epw)
    dst = edge_index[1].astype(jnp.int32).reshape(NW, epw)
    if pad:
        spare = n_pad - n_nodes
        fake = (n_nodes
                + (jnp.arange(NW)[:, None] * 7 + jnp.arange(pad)[None, :])
                % spare).astype(jnp.int32)
        fsrc = ((jnp.arange(NW)[:, None] * 131 + jnp.arange(pad)[None, :])
                % n_nodes).astype(jnp.int32)
        src = jnp.concatenate([src, fsrc], axis=1)
        dst = jnp.concatenate([dst, fake], axis=1)
    src = src.reshape(NW, nchunk, CHUNK)
    dst = dst.reshape(NW, nchunk, CHUNK)
    deg_i = jnp.asarray(deg).astype(jnp.int32).reshape(n_nodes, 1)
    ws = W_self.astype(jnp.float32)
    wn = W_neigh.astype(jnp.float32)
    bs = b_self.astype(jnp.float32).reshape(1, d_out)
    bn = b_neigh.astype(jnp.float32).reshape(1, d_out)

    segsum = _segsum_fn(n_nodes, d_in, nchunk)
    dense = _dense_fn(n_nodes, d_in, d_out)

    outs = []
    for bi in range(b):
        xb = x[bi].astype(jnp.float32)
        feat, cnt = segsum(xb, src, dst)
        cnt_n = cnt.reshape(NC, -1)[:, :n_nodes, None]
        outs.append(dense(xb, feat, cnt_n, deg_i, ws, bs, wn, bn))
    return jnp.stack(outs, axis=0).astype(x.dtype)


# speedup vs baseline: 1.5948x; 1.1198x over previous
import functools

import jax
import jax.numpy as jnp
from jax import lax
from jax.experimental import pallas as pl
from jax.experimental.pallas import tpu as pltpu
from jax.experimental.pallas import tpu_sc as plsc

NC = 2
NS = 16
NW = NC * NS
LANES = 16
CHUNK = 112
ZROWS = 64


def _pad_rows(n_nodes: int, d: int) -> int:
    unit = max(NS * 8, d)
    return -(-n_nodes // unit) * unit


@functools.cache
def _segsum_fn(n_nodes: int, d: int, epw: int):
    assert epw % 8 == 0
    nfull = epw // CHUNK
    tail = epw - nfull * CHUNK
    assert nfull % 2 == 1 and nfull >= 5
    n_pad = _pad_rows(n_nodes, d)
    rpt = n_pad // NS
    crows = n_pad // d
    assert d & (d - 1) == 0
    dshift = d.bit_length() - 1

    mesh = plsc.VectorSubcoreMesh(
        core_axis_name="c", subcore_axis_name="s",
        num_cores=NC, num_subcores=NS)

    scratch = [
        pltpu.VMEM_SHARED((n_pad, d), jnp.float32),
        pltpu.VMEM_SHARED((crows, d), jnp.int32),
        pltpu.VMEM((CHUNK,), jnp.int32),
        pltpu.VMEM((CHUNK,), jnp.int32),
        pltpu.VMEM((CHUNK,), jnp.int32),
        pltpu.VMEM((CHUNK,), jnp.int32),
        pltpu.VMEM((CHUNK, d), jnp.float32),
        pltpu.VMEM((CHUNK, d), jnp.float32),
        pltpu.VMEM((crows, d), jnp.int32),
        pltpu.VMEM((crows,), jnp.int32),
    ] + [pltpu.SemaphoreType.DMA] * 8
    if tail:
        scratch += [pltpu.VMEM((tail,), jnp.int32),
                    pltpu.VMEM((tail,), jnp.int32)]

    @functools.partial(
        pl.kernel,
        compiler_params=pltpu.CompilerParams(needs_layout_passes=False),
        out_type=(
            jax.ShapeDtypeStruct((NC, n_pad, d), jnp.float32),
            jax.ShapeDtypeStruct((NC, crows, d), jnp.int32),
        ),
        mesh=mesh,
        scratch_types=scratch,
    )
    def segsum(x_hbm, src_hbm, dst_hbm, feat_hbm, cnt_hbm,
               acc, cacc, idx_s0, idx_s1, idx_d0, idx_d1, rows0, rows1,
               cnt, idx_id, ses0, ses1, sed0, sed1, sg0, sg1, ss0, ss1,
               *tail_bufs):
        c = lax.axis_index("c")
        s = lax.axis_index("s")
        wid = s * NC + c
        ebase = wid * epw
        idx_s = (idx_s0, idx_s1)
        idx_d = (idx_d0, idx_d1)
        rows = (rows0, rows1)
        ses = (ses0, ses1)
        sed = (sed0, sed1)
        sg = (sg0, sg1)
        ss = (ss0, ss1)

        def zrow(i, carry):
            for j in range(d // LANES):
                rows0[i, pl.ds(j * LANES, LANES)] = jnp.zeros(
                    (LANES,), jnp.float32)
            return carry
        lax.fori_loop(0, ZROWS, zrow, 0)

        def crow(i, carry):
            for j in range(d // LANES):
                cnt[i, pl.ds(j * LANES, LANES)] = jnp.zeros(
                    (LANES,), jnp.int32)
            return carry
        lax.fori_loop(0, crows, crow, 0)

        for j in range(crows // LANES):
            idx_id[pl.ds(j * LANES, LANES)] = (
                lax.iota(jnp.int32, LANES) + j * LANES)
        if crows % LANES:
            idx_id[pl.ds(crows - LANES, LANES)] = (
                lax.iota(jnp.int32, LANES) + (crows - LANES))

        zcopies = [
            pltpu.async_copy(rows0.at[pl.ds(0, ZROWS)],
                             acc.at[pl.ds(s * rpt + k * ZROWS, ZROWS)],
                             sg0)
            for k in range(rpt // ZROWS)]
        ztail = rpt % ZROWS
        if ztail:
            zcopies.append(pltpu.async_copy(
                rows0.at[pl.ds(0, ztail)],
                acc.at[pl.ds(s * rpt + rpt - ztail, ztail)], sg0))
        @pl.when(s == 0)
        def _clear_cacc():
            pltpu.sync_copy(cnt, cacc)
        for zc in zcopies:
            zc.wait()
        plsc.subcore_barrier()

        ones16 = jnp.ones((LANES,), jnp.int32)
        last = nfull - 1

        def start_idx_s(j, p):
            jc = jnp.minimum(j, last)
            pltpu.async_copy(
                src_hbm.at[pl.ds(ebase + jc * CHUNK, CHUNK)],
                idx_s[p], ses[p])

        def wait_idx_s(p):
            pltpu.make_async_copy(
                src_hbm.at[pl.ds(0, CHUNK)], idx_s[p], ses[p]).wait()

        def start_idx_d(j, p):
            pltpu.async_copy(
                dst_hbm.at[pl.ds(ebase + j * CHUNK, CHUNK)],
                idx_d[p], sed[p])

        def wait_idx_d(p):
            pltpu.make_async_copy(
                dst_hbm.at[pl.ds(0, CHUNK)], idx_d[p], sed[p]).wait()

        def start_gather(p):
            pltpu.async_copy(x_hbm.at[idx_s[p]], rows[p], sg[p])

        def wait_gather(p):
            pltpu.make_async_copy(x_hbm.at[idx_s[p]], rows[p],
                                  sg[p]).wait()

        def start_scatter(p):
            pltpu.async_copy(rows[p], acc.at[idx_d[p]], ss[p], add=True)

        def wait_scatter(p):
            pltpu.make_async_copy(rows[p], acc.at[idx_d[p]],
                                  ss[p]).wait()

        def hist(ref, n_vec):
            for v in range(n_vec):
                dv = ref[pl.ds(v * LANES, LANES)]
                plsc.addupdate_scatter(
                    cnt,
                    [lax.shift_right_logical(dv, dshift),
                     lax.bitwise_and(dv, d - 1)],
                    ones16)

        start_idx_s(0, 0)
        start_idx_d(0, 0)
        start_idx_s(1, 1)
        wait_idx_s(0)
        start_gather(0)
        wait_gather(0)
        wait_idx_d(0)
        start_scatter(0)
        start_idx_s(2, 0)
        hist(idx_d0, CHUNK // LANES)
        wait_idx_s(1)
        start_gather(1)
        start_idx_d(1, 1)
        wait_gather(1)
        wait_idx_d(1)
        start_scatter(1)
        start_idx_s(3, 1)
        hist(idx_d1, CHUNK // LANES)
        wait_scatter(0)
        start_idx_d(2, 0)
        wait_idx_s(0)
        start_gather(0)

        def body(t, carry):
            j0 = 2 * t
            wait_gather(0)
            wait_idx_d(0)
            start_scatter(0)
            start_idx_s(j0 + 2, 0)
            hist(idx_d0, CHUNK // LANES)
            wait_scatter(1)
            start_idx_d(j0 + 1, 1)
            wait_idx_s(1)
            start_gather(1)
            wait_gather(1)
            wait_idx_d(1)
            start_scatter(1)
            start_idx_s(j0 + 3, 1)
            hist(idx_d1, CHUNK // LANES)
            wait_scatter(0)
            start_idx_d(j0 + 2, 0)
            wait_idx_s(0)
            start_gather(0)
            return carry
        lax.fori_loop(1, (nfull - 1) // 2, body, 0)

        wait_gather(0)
        wait_idx_d(0)
        start_scatter(0)
        hist(idx_d0, CHUNK // LANES)
        wait_idx_s(1)
        wait_scatter(1)
        wait_scatter(0)

        if tail:
            idx_st, idx_dt = tail_bufs
            tbase = ebase + nfull * CHUNK
            pltpu.sync_copy(src_hbm.at[pl.ds(tbase, tail)], idx_st)
            pltpu.sync_copy(dst_hbm.at[pl.ds(tbase, tail)], idx_dt)
            pltpu.async_copy(
                x_hbm.at[idx_st], rows0.at[pl.ds(0, tail)], sg0).wait()
            pltpu.sync_copy(rows0.at[pl.ds(0, tail)],
                            acc.at[idx_dt], add=True)
            hist(idx_dt, tail // LANES)

        pltpu.sync_copy(cnt, cacc.at[idx_id], add=True)
        plsc.subcore_barrier()

        pltpu.sync_copy(acc.at[pl.ds(s * rpt, rpt)],
                        feat_hbm.at[c, pl.ds(s * rpt, rpt)])
        @pl.when(s == 0)
        def _dump_cnt():
            pltpu.sync_copy(cacc, cnt_hbm.at[c])

    return segsum


@functools.cache
def _dense_fn(n_nodes: int, d_in: int, d_out: int):
    blk = 2000
    grid = n_nodes // blk
    assert blk * grid == n_nodes

    dn = (((1,), (1,)), ((), ()))

    def body(x_ref, p_ref, cnt_ref, deg_ref, ws_ref, bs_ref, wn_ref,
             bn_ref, o_ref):
        ns = p_ref[0] + p_ref[1]
        cnt = (cnt_ref[0] + cnt_ref[1]).astype(jnp.float32)
        agg = (lax.dot_general(ns, wn_ref[...], dn,
                               preferred_element_type=jnp.float32)
               + cnt * bn_ref[...])
        denom = jnp.maximum(deg_ref[...].astype(jnp.float32), 1.0)
        z = (lax.dot_general(x_ref[...], ws_ref[...], dn,
                             preferred_element_type=jnp.float32)
             + bs_ref[...] + agg / denom)
        o_ref[...] = jnp.where(z >= 0.0, z, 0.1 * z)

    return pl.pallas_call(
        body,
        grid=(grid,),
        in_specs=[
            pl.BlockSpec((blk, d_in), lambda i: (i, 0)),
            pl.BlockSpec((NC, blk, d_in), lambda i: (0, i, 0)),
            pl.BlockSpec((NC, blk, 1), lambda i: (0, i, 0)),
            pl.BlockSpec((blk, 1), lambda i: (i, 0)),
            pl.BlockSpec((d_out, d_in), lambda i: (0, 0)),
            pl.BlockSpec((1, d_out), lambda i: (0, 0)),
            pl.BlockSpec((d_out, d_in), lambda i: (0, 0)),
            pl.BlockSpec((1, d_out), lambda i: (0, 0)),
        ],
        out_specs=pl.BlockSpec((blk, d_out), lambda i: (i, 0)),
        out_shape=jax.ShapeDtypeStruct((n_nodes, d_out), jnp.float32),
    )


def kernel(x, edge_index, deg, W_self, b_self, W_neigh, b_neigh):
    b, n_nodes, d_in = x.shape
    d_out = W_neigh.shape[0]
    n_edges = edge_index.shape[1]
    epw = n_edges // NW
    assert epw * NW == n_edges

    ei = jnp.asarray(edge_index).astype(jnp.int32)
    src = ei[0]
    dst = ei[1]
    deg_i = jnp.asarray(deg).astype(jnp.int32).reshape(n_nodes, 1)
    ws = W_self.astype(jnp.float32)
    wn = W_neigh.astype(jnp.float32)
    bs = b_self.astype(jnp.float32).reshape(1, d_out)
    bn = b_neigh.astype(jnp.float32).reshape(1, d_out)

    segsum = _segsum_fn(n_nodes, d_in, epw)
    dense = _dense_fn(n_nodes, d_in, d_out)

    outs = []
    for bi in range(b):
        xb = x[bi].astype(jnp.float32)
        feat, cnt = segsum(xb, src, dst)
        cnt_n = cnt.reshape(NC, -1)[:, :n_nodes, None]
        outs.append(dense(xb, feat, cnt_n, deg_i, ws, bs, wn, bn))
    return jnp.stack(outs, axis=0).astype(x.dtype)


# speedup vs baseline: 1.7678x; 1.1085x over previous
import functools

import jax
import jax.numpy as jnp
from jax import lax
from jax.experimental import pallas as pl
from jax.experimental.pallas import tpu as pltpu
from jax.experimental.pallas import tpu_sc as plsc

NC = 2
NS = 16
NW = NC * NS
LANES = 16
CHUNK = 128
ZROWS = 64


def _pad_rows(n_nodes: int, d: int) -> int:
    unit = max(NS * 8, d)
    return -(-n_nodes // unit) * unit


@functools.cache
def _segsum_fn(n_nodes: int, d: int, nblocks: int):
    nbase = nblocks // NW
    rem = nblocks % NW
    assert nbase % 2 == 0 and nbase >= 4
    n_pad = _pad_rows(n_nodes, d)
    rpt = n_pad // NS
    crows = n_pad // d
    assert d & (d - 1) == 0
    dshift = d.bit_length() - 1
    nvec = CHUNK // LANES

    mesh = plsc.VectorSubcoreMesh(
        core_axis_name="c", subcore_axis_name="s",
        num_cores=NC, num_subcores=NS)

    scratch = [
        pltpu.VMEM_SHARED((n_pad, d), jnp.float32),
        pltpu.VMEM_SHARED((crows, d), jnp.int32),
        pltpu.VMEM((2, CHUNK), jnp.int32),
        pltpu.VMEM((2, CHUNK), jnp.int32),
        pltpu.VMEM((CHUNK,), jnp.int32),
        pltpu.VMEM((CHUNK,), jnp.int32),
        pltpu.VMEM((CHUNK, d), jnp.float32),
        pltpu.VMEM((CHUNK, d), jnp.float32),
        pltpu.VMEM((crows, d), jnp.int32),
        pltpu.VMEM((crows,), jnp.int32),
    ] + [pltpu.SemaphoreType.DMA] * 6

    @functools.partial(
        pl.kernel,
        compiler_params=pltpu.CompilerParams(needs_layout_passes=False),
        out_type=(
            jax.ShapeDtypeStruct((NC, n_pad, d), jnp.float32),
            jax.ShapeDtypeStruct((NC, crows, d), jnp.int32),
        ),
        mesh=mesh,
        scratch_types=scratch,
    )
    def segsum(x_hbm, ei_hbm, feat_hbm, cnt_hbm,
               acc, cacc, buf0, buf1, sdix0, sdix1, rows0, rows1,
               cnt, idx_id, si0, si1, sg0, sg1, ss0, ss1):
        c = lax.axis_index("c")
        s = lax.axis_index("s")
        wid = s * NC + c
        buf = (buf0, buf1)
        sdix = (sdix0, sdix1)
        rows = (rows0, rows1)
        si = (si0, si1)
        sg = (sg0, sg1)
        ss = (ss0, ss1)

        ones16 = jnp.ones((LANES,), jnp.int32)
        lastb = nbase - 1

        def start_idx(t, p):
            tc_ = jnp.minimum(t, lastb)
            pltpu.async_copy(ei_hbm.at[tc_ * NW + wid], buf[p], si[p])

        def wait_idx(p):
            pltpu.make_async_copy(ei_hbm.at[0], buf[p], si[p]).wait()

        def start_gather(p):
            pltpu.async_copy(x_hbm.at[buf[p].at[0]], rows[p], sg[p])

        def wait_gather(p):
            pltpu.make_async_copy(x_hbm.at[buf[p].at[0]], rows[p],
                                  sg[p]).wait()

        def grab_dst(p):
            for v in range(nvec):
                sdix[p][pl.ds(v * LANES, LANES)] = (
                    buf[p][1, pl.ds(v * LANES, LANES)])

        def start_scatter(p):
            pltpu.async_copy(rows[p], acc.at[sdix[p]], ss[p], add=True)

        def wait_scatter(p):
            pltpu.make_async_copy(rows[p], acc.at[sdix[p]],
                                  ss[p]).wait()

        def hist(ref, n_vec):
            for v in range(n_vec):
                dv = ref[pl.ds(v * LANES, LANES)]
                plsc.addupdate_scatter(
                    cnt,
                    [lax.shift_right_logical(dv, dshift),
                     lax.bitwise_and(dv, d - 1)],
                    ones16)

        start_idx(0, 0)
        start_idx(1, 1)

        def zrow(i, carry):
            for j in range(d // LANES):
                rows0[i, pl.ds(j * LANES, LANES)] = jnp.zeros(
                    (LANES,), jnp.float32)
            return carry
        lax.fori_loop(0, ZROWS, zrow, 0)

        def crow(i, carry):
            for j in range(d // LANES):
                cnt[i, pl.ds(j * LANES, LANES)] = jnp.zeros(
                    (LANES,), jnp.int32)
            return carry
        lax.fori_loop(0, crows, crow, 0)

        for j in range(crows // LANES):
            idx_id[pl.ds(j * LANES, LANES)] = (
                lax.iota(jnp.int32, LANES) + j * LANES)
        if crows % LANES:
            idx_id[pl.ds(crows - LANES, LANES)] = (
                lax.iota(jnp.int32, LANES) + (crows - LANES))

        zcopies = [
            pltpu.async_copy(rows0.at[pl.ds(0, ZROWS)],
                             acc.at[pl.ds(s * rpt + k * ZROWS, ZROWS)],
                             sg0)
            for k in range(rpt // ZROWS)]
        ztail = rpt % ZROWS
        if ztail:
            zcopies.append(pltpu.async_copy(
                rows0.at[pl.ds(0, ztail)],
                acc.at[pl.ds(s * rpt + rpt - ztail, ztail)], sg0))
        @pl.when(s == 0)
        def _clear_cacc():
            pltpu.sync_copy(cnt, cacc)

        wait_idx(1)
        start_gather(1)
        for zc in zcopies:
            zc.wait()
        wait_idx(0)
        start_gather(0)
        plsc.subcore_barrier()

        wait_gather(0)
        grab_dst(0)
        start_scatter(0)
        start_idx(2, 0)
        hist(sdix0, nvec)
        wait_gather(1)
        grab_dst(1)
        start_scatter(1)
        start_idx(3, 1)
        hist(sdix1, nvec)
        wait_scatter(0)
        wait_idx(0)
        start_gather(0)

        def body(t, carry):
            wait_gather(0)
            grab_dst(0)
            start_scatter(0)
            start_idx(2 * t + 2, 0)
            hist(sdix0, nvec)
            wait_scatter(1)
            wait_idx(1)
            start_gather(1)
            wait_gather(1)
            grab_dst(1)
            start_scatter(1)
            start_idx(2 * t + 3, 1)
            hist(sdix1, nvec)
            wait_scatter(0)
            wait_idx(0)
            start_gather(0)
            return carry
        lax.fori_loop(1, nbase // 2, body, 0)

        wait_gather(0)
        wait_idx(1)
        wait_scatter(1)

        if rem:
            @pl.when(wid < rem)
            def _leftover():
                pltpu.sync_copy(ei_hbm.at[nbase * NW + wid], buf0)
                pltpu.async_copy(x_hbm.at[buf0.at[0]], rows0, sg0).wait()
                grab_dst(0)
                pltpu.sync_copy(rows0, acc.at[sdix0], add=True)
                hist(sdix0, nvec)

        pltpu.sync_copy(cnt, cacc.at[idx_id], add=True)
        plsc.subcore_barrier()

        pltpu.sync_copy(acc.at[pl.ds(s * rpt, rpt)],
                        feat_hbm.at[c, pl.ds(s * rpt, rpt)])
        @pl.when(s == 0)
        def _dump_cnt():
            pltpu.sync_copy(cacc, cnt_hbm.at[c])

    return segsum


@functools.cache
def _dense_fn(n_nodes: int, d_in: int, d_out: int):
    blk = 2000
    grid = n_nodes // blk
    assert blk * grid == n_nodes

    dn = (((1,), (1,)), ((), ()))

    def body(x_ref, p_ref, cnt_ref, deg_ref, ws_ref, bs_ref, wn_ref,
             bn_ref, o_ref):
        ns = p_ref[0] + p_ref[1]
        cnt = (cnt_ref[0] + cnt_ref[1]).astype(jnp.float32)
        agg = (lax.dot_general(ns, wn_ref[...], dn,
                               preferred_element_type=jnp.float32)
               + cnt * bn_ref[...])
        denom = jnp.maximum(deg_ref[...].astype(jnp.float32), 1.0)
        z = (lax.dot_general(x_ref[...], ws_ref[...], dn,
                             preferred_element_type=jnp.float32)
             + bs_ref[...] + agg / denom)
        o_ref[...] = jnp.where(z >= 0.0, z, 0.1 * z)

    return pl.pallas_call(
        body,
        grid=(grid,),
        in_specs=[
            pl.BlockSpec((blk, d_in), lambda i: (i, 0)),
            pl.BlockSpec((NC, blk, d_in), lambda i: (0, i, 0)),
            pl.BlockSpec((NC, blk, 1), lambda i: (0, i, 0)),
            pl.BlockSpec((blk, 1), lambda i: (i, 0)),
            pl.BlockSpec((d_out, d_in), lambda i: (0, 0)),
            pl.BlockSpec((1, d_out), lambda i: (0, 0)),
            pl.BlockSpec((d_out, d_in), lambda i: (0, 0)),
            pl.BlockSpec((1, d_out), lambda i: (0, 0)),
        ],
        out_specs=pl.BlockSpec((blk, d_out), lambda i: (i, 0)),
        out_shape=jax.ShapeDtypeStruct((n_nodes, d_out), jnp.float32),
    )


def kernel(x, edge_index, deg, W_self, b_self, W_neigh, b_neigh):
    b, n_nodes, d_in = x.shape
    d_out = W_neigh.shape[0]
    n_edges = edge_index.shape[1]
    assert n_edges % CHUNK == 0
    nblocks = n_edges // CHUNK

    ei = jnp.asarray(edge_index).astype(jnp.int32)
    ei3 = ei.reshape(2, nblocks, CHUNK).transpose(1, 0, 2)
    deg_i = jnp.asarray(deg).astype(jnp.int32).reshape(n_nodes, 1)
    ws = W_self.astype(jnp.float32)
    wn = W_neigh.astype(jnp.float32)
    bs = b_self.astype(jnp.float32).reshape(1, d_out)
    bn = b_neigh.astype(jnp.float32).reshape(1, d_out)

    segsum = _segsum_fn(n_nodes, d_in, nblocks)
    dense = _dense_fn(n_nodes, d_in, d_out)

    outs = []
    for bi in range(b):
        xb = x[bi].astype(jnp.float32)
        feat, cnt = segsum(xb, ei3)
        cnt_n = cnt.reshape(NC, -1)[:, :n_nodes, None]
        outs.append(dense(xb, feat, cnt_n, deg_i, ws, bs, wn, bn))
    return jnp.stack(outs, axis=0).astype(x.dtype)


# speedup vs baseline: 1.8402x; 1.0409x over previous
import functools

import jax
import jax.numpy as jnp
from jax import lax
from jax.experimental import pallas as pl
from jax.experimental.pallas import tpu as pltpu
from jax.experimental.pallas import tpu_sc as plsc

NC = 2
NS = 16
NW = NC * NS
LANES = 16
CHUNK = 128
ZROWS = 64


def _pad_rows(n_nodes: int, d: int) -> int:
    unit = max(NS * 8, d)
    return -(-n_nodes // unit) * unit


@functools.cache
def _segsum_fn(n_nodes: int, d: int, nblocks: int):
    nbase = nblocks // NW
    rem = nblocks % NW
    assert nbase % 2 == 0 and nbase >= 4
    n_pad = _pad_rows(n_nodes, d)
    rpt = n_pad // NS
    crows = n_pad // d
    assert d & (d - 1) == 0
    dshift = d.bit_length() - 1
    nvec = CHUNK // LANES

    mesh = plsc.VectorSubcoreMesh(
        core_axis_name="c", subcore_axis_name="s",
        num_cores=NC, num_subcores=NS)

    scratch = [
        pltpu.VMEM_SHARED((n_pad, d), jnp.float32),
        pltpu.VMEM_SHARED((crows, d), jnp.int32),
        pltpu.VMEM((2, CHUNK), jnp.int32),
        pltpu.VMEM((2, CHUNK), jnp.int32),
        pltpu.VMEM((CHUNK,), jnp.int32),
        pltpu.VMEM((CHUNK,), jnp.int32),
        pltpu.VMEM((CHUNK, d), jnp.float32),
        pltpu.VMEM((CHUNK, d), jnp.float32),
        pltpu.VMEM((crows, d), jnp.int32),
        pltpu.VMEM((crows,), jnp.int32),
    ] + [pltpu.SemaphoreType.DMA] * 6

    @functools.partial(
        pl.kernel,
        compiler_params=pltpu.CompilerParams(needs_layout_passes=False),
        out_type=(
            jax.ShapeDtypeStruct((NC, n_pad, d), jnp.float32),
            jax.ShapeDtypeStruct((NC, crows, d), jnp.int32),
        ),
        mesh=mesh,
        scratch_types=scratch,
    )
    def segsum(x_hbm, ei_hbm, feat_hbm, cnt_hbm,
               acc, cacc, buf0, buf1, sdix0, sdix1, rows0, rows1,
               cnt, idx_id, si0, si1, sg0, sg1, ss0, ss1):
        c = lax.axis_index("c")
        s = lax.axis_index("s")
        wid = s * NC + c
        buf = (buf0, buf1)
        sdix = (sdix0, sdix1)
        rows = (rows0, rows1)
        si = (si0, si1)
        sg = (sg0, sg1)
        ss = (ss0, ss1)

        ones16 = jnp.ones((LANES,), jnp.int32)
        lastb = nbase - 1

        def start_idx(t, p):
            tc_ = jnp.minimum(t, lastb)
            pltpu.async_copy(ei_hbm.at[tc_ * NW + wid], buf[p], si[p])

        def wait_idx(p):
            pltpu.make_async_copy(ei_hbm.at[0], buf[p], si[p]).wait()

        def start_gather(p):
            pltpu.async_copy(x_hbm.at[buf[p].at[0]], rows[p], sg[p])

        def wait_gather(p):
            pltpu.make_async_copy(x_hbm.at[buf[p].at[0]], rows[p],
                                  sg[p]).wait()

        def grab_dst(p):
            for v in range(nvec):
                sdix[p][pl.ds(v * LANES, LANES)] = (
                    buf[p][1, pl.ds(v * LANES, LANES)])

        def start_scatter(p):
            pltpu.async_copy(rows[p], acc.at[sdix[p]], ss[p], add=True)

        def wait_scatter(p):
            pltpu.make_async_copy(rows[p], acc.at[sdix[p]],
                                  ss[p]).wait()

        def hist(ref, n_vec):
            for v in range(n_vec):
                dv = ref[pl.ds(v * LANES, LANES)]
                plsc.addupdate_scatter(
                    cnt,
                    [lax.shift_right_logical(dv, dshift),
                     lax.bitwise_and(dv, d - 1)],
                    ones16)

        start_idx(0, 0)
        start_idx(1, 1)

        def zrow(i, carry):
            for j in range(d // LANES):
                rows0[i, pl.ds(j * LANES, LANES)] = jnp.zeros(
                    (LANES,), jnp.float32)
            return carry
        lax.fori_loop(0, ZROWS, zrow, 0)

        def crow(i, carry):
            for j in range(d // LANES):
                cnt[i, pl.ds(j * LANES, LANES)] = jnp.zeros(
                    (LANES,), jnp.int32)
            return carry
        lax.fori_loop(0, crows, crow, 0)

        for j in range(crows // LANES):
            idx_id[pl.ds(j * LANES, LANES)] = (
                lax.iota(jnp.int32, LANES) + j * LANES)
        if crows % LANES:
            idx_id[pl.ds(crows - LANES, LANES)] = (
                lax.iota(jnp.int32, LANES) + (crows - LANES))

        zcopies = [
            pltpu.async_copy(rows0.at[pl.ds(0, ZROWS)],
                             acc.at[pl.ds(s * rpt + k * ZROWS, ZROWS)],
                             sg0)
            for k in range(rpt // ZROWS)]
        ztail = rpt % ZROWS
        if ztail:
            zcopies.append(pltpu.async_copy(
                rows0.at[pl.ds(0, ztail)],
                acc.at[pl.ds(s * rpt + rpt - ztail, ztail)], sg0))
        @pl.when(s == 0)
        def _clear_cacc():
            pltpu.sync_copy(cnt, cacc)

        wait_idx(1)
        start_gather(1)
        for zc in zcopies:
            zc.wait()
        wait_idx(0)
        start_gather(0)
        plsc.subcore_barrier()

        wait_gather(0)
        grab_dst(0)
        start_scatter(0)
        start_idx(2, 0)
        hist(sdix0, nvec)
        wait_gather(1)
        grab_dst(1)
        start_scatter(1)
        start_idx(3, 1)
        hist(sdix1, nvec)
        wait_scatter(0)
        wait_idx(0)
        start_gather(0)

        def body(t, carry):
            wait_gather(0)
            grab_dst(0)
            start_scatter(0)
            start_idx(2 * t + 2, 0)
            hist(sdix0, nvec)
            wait_scatter(1)
            wait_idx(1)
            start_gather(1)
            wait_gather(1)
            grab_dst(1)
            start_scatter(1)
            start_idx(2 * t + 3, 1)
            hist(sdix1, nvec)
            wait_scatter(0)
            wait_idx(0)
            start_gather(0)
            return carry
        lax.fori_loop(1, nbase // 2, body, 0)

        wait_gather(0)
        wait_idx(1)
        wait_scatter(1)

        if rem:
            @pl.when(wid < rem)
            def _leftover():
                pltpu.sync_copy(ei_hbm.at[nbase * NW + wid], buf0)
                pltpu.async_copy(x_hbm.at[buf0.at[0]], rows0, sg0).wait()
                grab_dst(0)
                pltpu.sync_copy(rows0, acc.at[sdix0], add=True)
                hist(sdix0, nvec)

        pltpu.sync_copy(cnt, cacc.at[idx_id], add=True)
        plsc.subcore_barrier()

        pltpu.sync_copy(acc.at[pl.ds(s * rpt, rpt)],
                        feat_hbm.at[c, pl.ds(s * rpt, rpt)])
        @pl.when(s == 0)
        def _dump_cnt():
            pltpu.sync_copy(cacc, cnt_hbm.at[c])

    return segsum


@functools.cache
def _dense_fn(n_nodes: int, d_in: int, d_out: int):
    blk = 2048
    grid = -(-n_nodes // blk)

    dn = (((1,), (1,)), ((), ()))

    def body(x_ref, p_ref, cnt_ref, deg_ref, ws_ref, bs_ref, wn_ref,
             bn_ref, o_ref):
        ns = p_ref[0] + p_ref[1]
        cnt = jnp.swapaxes(cnt_ref[0] + cnt_ref[1],
                           0, 1).astype(jnp.float32)
        agg = (lax.dot_general(ns, wn_ref[...], dn,
                               preferred_element_type=jnp.float32)
               + cnt * bn_ref[...])
        deg_col = jnp.swapaxes(deg_ref[...], 0, 1)
        denom = jnp.maximum(deg_col.astype(jnp.float32), 1.0)
        z = (lax.dot_general(x_ref[...], ws_ref[...], dn,
                             preferred_element_type=jnp.float32)
             + bs_ref[...] + agg / denom)
        o_ref[...] = jnp.where(z >= 0.0, z, 0.1 * z)

    return pl.pallas_call(
        body,
        grid=(grid,),
        in_specs=[
            pl.BlockSpec((blk, d_in), lambda i: (i, 0)),
            pl.BlockSpec((NC, blk, d_in), lambda i: (0, i, 0)),
            pl.BlockSpec((NC, 1, blk), lambda i: (0, 0, i)),
            pl.BlockSpec((1, blk), lambda i: (0, i)),
            pl.BlockSpec((d_out, d_in), lambda i: (0, 0)),
            pl.BlockSpec((1, d_out), lambda i: (0, 0)),
            pl.BlockSpec((d_out, d_in), lambda i: (0, 0)),
            pl.BlockSpec((1, d_out), lambda i: (0, 0)),
        ],
        out_specs=pl.BlockSpec((blk, d_out), lambda i: (i, 0)),
        out_shape=jax.ShapeDtypeStruct((n_nodes, d_out), jnp.float32),
    )


def kernel(x, edge_index, deg, W_self, b_self, W_neigh, b_neigh):
    b, n_nodes, d_in = x.shape
    d_out = W_neigh.shape[0]
    n_edges = edge_index.shape[1]
    assert n_edges % CHUNK == 0
    nblocks = n_edges // CHUNK

    ei = jnp.asarray(edge_index).astype(jnp.int32)
    ei3 = ei.reshape(2, nblocks, CHUNK).transpose(1, 0, 2)
    deg_i = jnp.asarray(deg).astype(jnp.int32).reshape(1, n_nodes)
    ws = W_self.astype(jnp.float32)
    wn = W_neigh.astype(jnp.float32)
    bs = b_self.astype(jnp.float32).reshape(1, d_out)
    bn = b_neigh.astype(jnp.float32).reshape(1, d_out)

    segsum = _segsum_fn(n_nodes, d_in, nblocks)
    dense = _dense_fn(n_nodes, d_in, d_out)

    outs = []
    for bi in range(b):
        xb = x[bi].astype(jnp.float32)
        feat, cnt = segsum(xb, ei3)
        cnt_n = cnt.reshape(NC, -1)[:, None, :n_nodes]
        outs.append(dense(xb, feat, cnt_n, deg_i, ws, bs, wn, bn))
    return jnp.stack(outs, axis=0).astype(x.dtype)
